# Initial kernel scaffold; baseline (speedup 1.0000x reference)
import jax
import jax.numpy as jnp
from jax.experimental import pallas as pl


def kernel(feature, indices, tables):
    raise NotImplementedError("write your pallas kernel here")


# speedup vs baseline: 1.0046x; 1.0046x over previous
import functools

import jax
import jax.numpy as jnp
from jax import lax
from jax.experimental import pallas as pl
from jax.experimental.pallas import tpu as pltpu
from jax.experimental.pallas import tpu_sc as plsc

N_FIELDS = 26
VOCAB = 100000
N_DIM = 32
BATCH = 4096
L = 200

NC = 2
NS = 16
NW = NC * NS
B_FLAT = BATCH * N_FIELDS
ROWS_PER_W = B_FLAT // NW
CHUNK = 128
N_CHUNKS = ROWS_PER_W // CHUNK


def _gather_body(tab_hbm, idx_hbm, out_hbm, idx_v, rows_v, sem):
    wid = lax.axis_index("s") * NC + lax.axis_index("c")
    pltpu.sync_copy(idx_hbm.at[wid], idx_v)
    handles = []
    for j in range(N_CHUNKS):
        handles.append(
            pltpu.async_copy(
                tab_hbm.at[idx_v.at[j]],
                rows_v.at[pl.ds(j * CHUNK, CHUNK)],
                sem,
            )
        )
    for h in handles:
        h.wait()
    pltpu.sync_copy(rows_v, out_hbm.at[pl.ds(wid * ROWS_PER_W, ROWS_PER_W)])


_gather = functools.partial(
    pl.kernel,
    mesh=plsc.VectorSubcoreMesh(core_axis_name="c", subcore_axis_name="s"),
    compiler_params=pltpu.CompilerParams(use_tc_tiling_on_sc=False),
    out_type=jax.ShapeDtypeStruct((B_FLAT, N_DIM), jnp.float32),
    scratch_types=[
        pltpu.VMEM((N_CHUNKS, CHUNK), jnp.int32),
        pltpu.VMEM((ROWS_PER_W, N_DIM), jnp.float32),
        pltpu.SemaphoreType.DMA,
    ],
)(_gather_body)


def kernel(feature, indices, tables):
    tab_flat = tables.reshape(N_FIELDS * VOCAB, N_DIM)
    flat_idx = indices.astype(jnp.int32) + (
        jnp.arange(N_FIELDS, dtype=jnp.int32) * VOCAB
    )[None, :]
    flat_idx = flat_idx.reshape(NW, N_CHUNKS, CHUNK)
    embeds = _gather(tab_flat, flat_idx).reshape(BATCH, N_FIELDS, N_DIM)
    return jnp.concatenate([embeds, feature], axis=1)
